# Initial kernel scaffold; baseline (speedup 1.0000x reference)
import jax
import jax.numpy as jnp
from jax.experimental import pallas as pl


def kernel(input_ids, word_emb, pos_emb, tt_emb, ln_weight, ln_bias):
    raise NotImplementedError("write your pallas kernel here")


# speedup vs baseline: 3.8281x; 3.8281x over previous
import dataclasses
import functools

import jax
import jax.numpy as jnp
from jax import lax
from jax.experimental import pallas as pl
from jax.experimental.pallas import tpu as pltpu
from jax.experimental.pallas import tpu_sc as plsc

H = 64
L_SEQ = 200
NC, NS = 2, 16
NW = NC * NS
LANES = 16
VPR = H // LANES
EPS = 1e-12
GCHUNKS = ((0, 104), (104, 96))


def _rsqrt_vec(a):
    ai = plsc.bitcast(a, jnp.int32)
    yi = jnp.int32(0x5F3759DF) - lax.shift_right_logical(ai, 1)
    y = plsc.bitcast(yi, jnp.float32)
    h = a * 0.5
    for _ in range(3):
        y = y * (1.5 - h * y * y)
    return y


def _sc_embed_ln(ids_flat, word_emb, pos_emb, tt_emb, ln_w, ln_b):
    n = ids_flat.shape[0]
    seqs = n // L_SEQ
    seqs_per_w = seqs // NW
    ids_per_w = seqs_per_w * L_SEQ
    mesh = plsc.VectorSubcoreMesh(core_axis_name="c", subcore_axis_name="s")
    cp = pltpu.CompilerParams()
    if "needs_layout_passes" in pltpu.CompilerParams.__dataclass_fields__:
        cp = dataclasses.replace(cp, needs_layout_passes=False)
    if "use_tc_tiling_on_sc" in pltpu.CompilerParams.__dataclass_fields__:
        cp = dataclasses.replace(cp, use_tc_tiling_on_sc=False)

    @functools.partial(
        pl.kernel,
        out_type=jax.ShapeDtypeStruct((n, H), jnp.float32),
        mesh=mesh,
        compiler_params=cp,
        scratch_types=[
            pltpu.VMEM((ids_per_w,), jnp.int32),
            pltpu.VMEM((2, L_SEQ, H), jnp.float32),
            pltpu.VMEM((2, L_SEQ, H), jnp.float32),
            pltpu.VMEM((L_SEQ, H), jnp.float32),
            pltpu.VMEM((H,), jnp.float32),
            pltpu.VMEM((H,), jnp.float32),
            pltpu.VMEM((1, H), jnp.float32),
            pltpu.SemaphoreType.DMA,
            pltpu.SemaphoreType.DMA,
            pltpu.SemaphoreType.DMA,
            pltpu.SemaphoreType.DMA,
        ],
    )
    def k(ids_hbm, word_hbm, pos_hbm, tt_hbm, w_hbm, b_hbm, out_hbm,
          idx_v, rows_v, outb_v, pos_v, w_v, b_v, tt_v,
          gsem0, gsem1, osem0, osem1, *_):
        gsems = (gsem0, gsem1)
        osems = (osem0, osem1)
        wid = lax.axis_index("s") * NC + lax.axis_index("c")
        base0 = wid * ids_per_w

        pltpu.sync_copy(ids_hbm.at[pl.ds(base0, ids_per_w)], idx_v)
        pltpu.sync_copy(pos_hbm.at[pl.ds(0, L_SEQ)], pos_v)
        pltpu.sync_copy(tt_hbm.at[pl.ds(0, 1)], tt_v)
        pltpu.sync_copy(w_hbm, w_v)
        pltpu.sync_copy(b_hbm, b_v)

        @pl.loop(0, L_SEQ)
        def _(i):
            for kx in range(VPR):
                sl = pl.ds(kx * LANES, LANES)
                pos_v[i, sl] = pos_v[i, sl] + tt_v[0, sl]

        def gather_descs(g, buf):
            for off, sz in GCHUNKS:
                yield (word_hbm.at[idx_v.at[pl.ds(g * L_SEQ + off, sz)]],
                       rows_v.at[buf].at[pl.ds(off, sz)],
                       gsems[buf])

        def start_gather(g, buf):
            for src, dst, sem in gather_descs(g, buf):
                pltpu.async_copy(src, dst, sem)

        def wait_gather(g, buf):
            for src, dst, sem in gather_descs(g, buf):
                pltpu.make_async_copy(src, dst, sem).wait()

        def out_desc(g, buf):
            return (outb_v.at[buf], out_hbm.at[pl.ds(base0 + g * L_SEQ, L_SEQ)],
                    osems[buf])

        def norm_rows(buf):
            @pl.loop(0, L_SEQ, step=2)
            def _(i):
                for r in range(2):
                    row = i + r
                    x = [rows_v[buf, row, pl.ds(kx * LANES, LANES)]
                         + pos_v[row, pl.ds(kx * LANES, LANES)]
                         for kx in range(VPR)]
                    tot = x[0] + x[1] + x[2] + x[3]
                    u = jnp.sum(tot) * (1.0 / H)
                    c = [xv - u for xv in x]
                    sq = (c[0] * c[0] + c[1] * c[1]
                          + c[2] * c[2] + c[3] * c[3])
                    ss = jnp.sum(sq) * (1.0 / H) + EPS
                    inv = _rsqrt_vec(jnp.broadcast_to(ss, (LANES,)))
                    for kx in range(VPR):
                        sl = pl.ds(kx * LANES, LANES)
                        outb_v[buf, row, sl] = c[kx] * inv * w_v[sl] + b_v[sl]

        for buf in range(2):
            start_gather(buf, buf)

        @pl.loop(0, seqs_per_w // 2)
        def _(it):
            for buf in range(2):
                g = it * 2 + buf
                wait_gather(g, buf)

                @pl.when(it > 0)
                def _():
                    src, dst, sem = out_desc(g - 2, buf)
                    pltpu.make_async_copy(src, dst, sem).wait()

                norm_rows(buf)
                pltpu.async_copy(*out_desc(g, buf))

                @pl.when(g + 2 < seqs_per_w)
                def _():
                    start_gather(g + 2, buf)

        for buf in range(2):
            src, dst, sem = out_desc(seqs_per_w - 2 + buf, buf)
            pltpu.make_async_copy(src, dst, sem).wait()

    return k(ids_flat, word_emb, pos_emb, tt_emb, ln_w, ln_b)


def kernel(input_ids, word_emb, pos_emb, tt_emb, ln_weight, ln_bias):
    b, l = input_ids.shape
    ids_flat = input_ids.reshape(-1).astype(jnp.int32)
    out = _sc_embed_ln(ids_flat, word_emb, pos_emb, tt_emb, ln_weight, ln_bias)
    return out.reshape(b, l, H)


# speedup vs baseline: 4.8348x; 1.2630x over previous
import dataclasses
import functools

import jax
import jax.numpy as jnp
from jax import lax
from jax.experimental import pallas as pl
from jax.experimental.pallas import tpu as pltpu
from jax.experimental.pallas import tpu_sc as plsc

H = 64
L_SEQ = 200
NC, NS = 2, 16
NW = NC * NS
LANES = 16
VPR = H // LANES
EPS = 1e-12
GCHUNKS = ((0, 104), (104, 96))


def _rsqrt_vec(a):
    ai = plsc.bitcast(a, jnp.int32)
    yi = jnp.int32(0x5F3759DF) - lax.shift_right_logical(ai, 1)
    y = plsc.bitcast(yi, jnp.float32)
    h = a * 0.5
    for _ in range(2):
        y = y * (1.5 - h * y * y)
    return y


def _sc_embed_ln(ids_flat, word_emb, pos_emb, tt_emb, ln_w, ln_b):
    n = ids_flat.shape[0]
    seqs = n // L_SEQ
    seqs_per_w = seqs // NW
    ids_per_w = seqs_per_w * L_SEQ
    mesh = plsc.VectorSubcoreMesh(core_axis_name="c", subcore_axis_name="s")
    cp = pltpu.CompilerParams()
    if "needs_layout_passes" in pltpu.CompilerParams.__dataclass_fields__:
        cp = dataclasses.replace(cp, needs_layout_passes=False)
    if "use_tc_tiling_on_sc" in pltpu.CompilerParams.__dataclass_fields__:
        cp = dataclasses.replace(cp, use_tc_tiling_on_sc=False)

    @functools.partial(
        pl.kernel,
        out_type=jax.ShapeDtypeStruct((n, H), jnp.float32),
        mesh=mesh,
        compiler_params=cp,
        scratch_types=[
            pltpu.VMEM((ids_per_w,), jnp.int32),
            pltpu.VMEM((2, L_SEQ, H), jnp.float32),
            pltpu.VMEM((2, L_SEQ, H), jnp.float32),
            pltpu.VMEM((L_SEQ, H), jnp.float32),
            pltpu.VMEM((H,), jnp.float32),
            pltpu.VMEM((H,), jnp.float32),
            pltpu.VMEM((1, H), jnp.float32),
            pltpu.SemaphoreType.DMA,
            pltpu.SemaphoreType.DMA,
            pltpu.SemaphoreType.DMA,
            pltpu.SemaphoreType.DMA,
        ],
    )
    def k(ids_hbm, word_hbm, pos_hbm, tt_hbm, w_hbm, b_hbm, out_hbm,
          idx_v, rows_v, outb_v, pos_v, w_v, b_v, tt_v,
          gsem0, gsem1, osem0, osem1, *_):
        gsems = (gsem0, gsem1)
        osems = (osem0, osem1)
        wid = lax.axis_index("s") * NC + lax.axis_index("c")
        base0 = wid * ids_per_w

        pltpu.sync_copy(ids_hbm.at[pl.ds(base0, ids_per_w)], idx_v)
        pltpu.sync_copy(pos_hbm.at[pl.ds(0, L_SEQ)], pos_v)
        pltpu.sync_copy(tt_hbm.at[pl.ds(0, 1)], tt_v)
        pltpu.sync_copy(w_hbm, w_v)
        pltpu.sync_copy(b_hbm, b_v)

        @pl.loop(0, L_SEQ)
        def _(i):
            for kx in range(VPR):
                sl = pl.ds(kx * LANES, LANES)
                pos_v[i, sl] = pos_v[i, sl] + tt_v[0, sl]

        def gather_descs(g, buf):
            for off, sz in GCHUNKS:
                yield (word_hbm.at[idx_v.at[pl.ds(g * L_SEQ + off, sz)]],
                       rows_v.at[buf].at[pl.ds(off, sz)],
                       gsems[buf])

        def start_gather(g, buf):
            for src, dst, sem in gather_descs(g, buf):
                pltpu.async_copy(src, dst, sem)

        def wait_gather(g, buf):
            for src, dst, sem in gather_descs(g, buf):
                pltpu.make_async_copy(src, dst, sem).wait()

        def out_desc(g, buf):
            return (outb_v.at[buf], out_hbm.at[pl.ds(base0 + g * L_SEQ, L_SEQ)],
                    osems[buf])

        def norm_rows(buf):
            wv = [w_v[pl.ds(kx * LANES, LANES)] for kx in range(VPR)]
            bv = [b_v[pl.ds(kx * LANES, LANES)] for kx in range(VPR)]

            @pl.loop(0, L_SEQ, step=4)
            def _(i):
                for r in range(4):
                    row = i + r
                    x = [rows_v[buf, row, pl.ds(kx * LANES, LANES)]
                         + pos_v[row, pl.ds(kx * LANES, LANES)]
                         for kx in range(VPR)]
                    s1 = (x[0] + x[1]) + (x[2] + x[3])
                    s2 = ((x[0] * x[0] + x[1] * x[1])
                          + (x[2] * x[2] + x[3] * x[3]))
                    u = jnp.sum(s1) * (1.0 / H)
                    var = jnp.sum(s2) * (1.0 / H) - u * u + EPS
                    inv = _rsqrt_vec(jnp.broadcast_to(var, (LANES,)))
                    for kx in range(VPR):
                        outb_v[buf, row, pl.ds(kx * LANES, LANES)] = (
                            (x[kx] - u) * inv * wv[kx] + bv[kx])

        for buf in range(2):
            start_gather(buf, buf)

        @pl.loop(0, seqs_per_w // 2)
        def _(it):
            for buf in range(2):
                g = it * 2 + buf
                wait_gather(g, buf)

                @pl.when(it > 0)
                def _():
                    src, dst, sem = out_desc(g - 2, buf)
                    pltpu.make_async_copy(src, dst, sem).wait()

                norm_rows(buf)
                pltpu.async_copy(*out_desc(g, buf))

                @pl.when(g + 2 < seqs_per_w)
                def _():
                    start_gather(g + 2, buf)

        for buf in range(2):
            src, dst, sem = out_desc(seqs_per_w - 2 + buf, buf)
            pltpu.make_async_copy(src, dst, sem).wait()

    return k(ids_flat, word_emb, pos_emb, tt_emb, ln_w, ln_b)


def kernel(input_ids, word_emb, pos_emb, tt_emb, ln_weight, ln_bias):
    b, l = input_ids.shape
    ids_flat = input_ids.reshape(-1).astype(jnp.int32)
    out = _sc_embed_ln(ids_flat, word_emb, pos_emb, tt_emb, ln_weight, ln_bias)
    return out.reshape(b, l, H)
